# baseline (speedup 1.0000x reference)
import jax
import jax.numpy as jnp
from jax.experimental import pallas as pl
from jax.experimental.pallas import tpu as pltpu

_B = 16384
_NH = 256
_NM = 1024
_NC = 128
_NMPC = 8
_EPS = 1e-4

_KA = 2 * _NH
_BLK = 256
_NBLK = _B // _BLK


def _loss_kernel(z_ref, y_ref, m2_ref, acc_ref, zaug_ref, maug_ref, msqe_ref):
    i = pl.program_id(0)

    @pl.when(i == 0)
    def _():
        m2 = m2_ref[...].astype(jnp.float32)
        msqe_ref[...] = 0.25 * jnp.sum(m2 * m2, axis=1, keepdims=True).T + _EPS
        maug_ref[:, 0:_NH] = jnp.ones((_NM, _NH), jnp.bfloat16)
        maug_ref[:, _NH:_KA] = m2_ref[...]

    zb = z_ref[...]
    zaug_ref[:, 0:_NH] = zb * zb
    zaug_ref[:, _NH:_KA] = zb
    zaug = zaug_ref[...]

    persum = jnp.zeros((_BLK, _NC), jnp.float32)
    for k in range(_NMPC):
        sl = pl.ds(k * _NC, _NC)
        dk = jax.lax.dot_general(
            zaug, maug_ref[sl, :], (((1,), (1,)), ((), ())),
            preferred_element_type=jnp.float32,
        )
        den = dk + msqe_ref[:, sl]
        r = 1.0 + (1.0 - _EPS) / den
        r2 = r * r
        persum = persum + r2 * r2 * r

    tot = jnp.sum(persum, axis=1)
    cls = jax.lax.broadcasted_iota(jnp.int32, (_BLK, _NC), 1)
    yb = y_ref[0, 0, :]
    pos = jnp.sum(jnp.where(cls == yb[:, None], persum, 0.0), axis=1)
    part = jnp.sum(jnp.log(tot) - jnp.log(pos)).reshape(1, 1)

    @pl.when(i == 0)
    def _():
        acc_ref[...] = part

    @pl.when(i != 0)
    def _():
        acc_ref[...] += part


def kernel(z, y, motif_vector):
    y3 = y.reshape(_NBLK, 1, _BLK)
    m2 = -2.0 * motif_vector.reshape(_NC, _NMPC, _NH).transpose(1, 0, 2)
    m2 = m2.reshape(_NM, _NH)
    acc = pl.pallas_call(
        _loss_kernel,
        grid=(_NBLK,),
        in_specs=[
            pl.BlockSpec((_BLK, _NH), lambda i: (i, 0)),
            pl.BlockSpec((1, 1, _BLK), lambda i: (i, 0, 0)),
            pl.BlockSpec((_NM, _NH), lambda i: (0, 0)),
        ],
        out_specs=pl.BlockSpec((1, 1), lambda i: (0, 0)),
        out_shape=jax.ShapeDtypeStruct((1, 1), jnp.float32),
        scratch_shapes=[
            pltpu.VMEM((_BLK, _KA), jnp.bfloat16),
            pltpu.VMEM((_NM, _KA), jnp.bfloat16),
            pltpu.VMEM((1, _NM), jnp.float32),
        ],
    )(z.astype(jnp.bfloat16), y3, m2.astype(jnp.bfloat16))
    return acc[0, 0] / _B


# speedup vs baseline: 1.1202x; 1.1202x over previous
import jax
import jax.numpy as jnp
from jax.experimental import pallas as pl
from jax.experimental.pallas import tpu as pltpu

_B = 16384
_NH = 256
_NM = 1024
_NC = 128
_NMPC = 8
_EPS = 1e-4

_KA = 2 * _NH
_BLK = 256
_NBLK = _B // _BLK


def _loss_kernel(z_ref, y_ref, m2_ref, acc_ref, zaug_ref, maug_ref, msqe_ref):
    i = pl.program_id(0)

    @pl.when(i == 0)
    def _():
        m2 = m2_ref[...]
        msqe_ref[...] = 0.25 * jnp.sum(m2 * m2, axis=1, keepdims=True).T + _EPS
        maug_ref[:, 0:_NH] = jnp.ones((_NM, _NH), jnp.bfloat16)
        maug_ref[:, _NH:_KA] = m2.astype(jnp.bfloat16)

    zb = z_ref[...]
    zaug_ref[:, 0:_NH] = (zb * zb).astype(jnp.bfloat16)
    zaug_ref[:, _NH:_KA] = zb.astype(jnp.bfloat16)
    zaug = zaug_ref[...]

    persum = jnp.zeros((_BLK, _NC), jnp.float32)
    for k in range(_NMPC):
        sl = pl.ds(k * _NC, _NC)
        dk = jax.lax.dot_general(
            zaug, maug_ref[sl, :], (((1,), (1,)), ((), ())),
            preferred_element_type=jnp.float32,
        )
        den = dk + msqe_ref[:, sl]
        r = 1.0 + (1.0 - _EPS) / den
        r2 = r * r
        persum = persum + r2 * r2 * r

    tot = jnp.sum(persum, axis=1)
    cls = jax.lax.broadcasted_iota(jnp.int32, (_BLK, _NC), 1)
    yb = y_ref[0, 0, :]
    pos = jnp.sum(jnp.where(cls == yb[:, None], persum, 0.0), axis=1)
    part = jnp.sum(jnp.log(tot) - jnp.log(pos)).reshape(1, 1)

    @pl.when(i == 0)
    def _():
        acc_ref[...] = part

    @pl.when(i != 0)
    def _():
        acc_ref[...] += part


def kernel(z, y, motif_vector):
    y3 = y.reshape(_NBLK, 1, _BLK)
    m2 = -2.0 * motif_vector.reshape(_NC, _NMPC, _NH).transpose(1, 0, 2)
    m2 = m2.reshape(_NM, _NH)
    acc = pl.pallas_call(
        _loss_kernel,
        grid=(_NBLK,),
        in_specs=[
            pl.BlockSpec((_BLK, _NH), lambda i: (i, 0)),
            pl.BlockSpec((1, 1, _BLK), lambda i: (i, 0, 0)),
            pl.BlockSpec((_NM, _NH), lambda i: (0, 0)),
        ],
        out_specs=pl.BlockSpec((1, 1), lambda i: (0, 0)),
        out_shape=jax.ShapeDtypeStruct((1, 1), jnp.float32),
        scratch_shapes=[
            pltpu.VMEM((_BLK, _KA), jnp.bfloat16),
            pltpu.VMEM((_NM, _KA), jnp.bfloat16),
            pltpu.VMEM((1, _NM), jnp.float32),
        ],
    )(z, y3, m2)
    return acc[0, 0] / _B


# speedup vs baseline: 1.8168x; 1.6219x over previous
import jax
import jax.numpy as jnp
from jax.experimental import pallas as pl
from jax.experimental.pallas import tpu as pltpu

_B = 16384
_NH = 256
_NM = 1024
_NC = 128
_NMPC = 8
_EPS = 1e-4

_BLK = 512
_NBLK = _B // _BLK


def _loss_kernel(z_ref, y_ref, m2_ref, acc_ref, msqe_ref):
    i = pl.program_id(0)

    @pl.when(i == 0)
    def _():
        m2 = m2_ref[...]
        msqe_ref[...] = 0.25 * jnp.sum(m2 * m2, axis=1, keepdims=True).T + _EPS

    zb = z_ref[...]
    zsq = jnp.sum(zb * zb, axis=1, keepdims=True)
    xp2 = jax.lax.dot_general(
        zb, m2_ref[...], (((1,), (1,)), ((), ())),
        preferred_element_type=jnp.float32,
    )
    den = zsq + msqe_ref[...] + xp2
    r = 1.0 + (1.0 - _EPS) / den
    r2 = r * r
    sim = r2 * r2 * r
    persum = sim[:, 0:_NC]
    for k in range(1, _NMPC):
        persum = persum + sim[:, k * _NC:(k + 1) * _NC]
    tot = jnp.sum(persum, axis=1)
    cls = jax.lax.broadcasted_iota(jnp.int32, (_BLK, _NC), 1)
    yb = y_ref[0, 0, :]
    pos = jnp.sum(jnp.where(cls == yb[:, None], persum, 0.0), axis=1)
    part = jnp.sum(jnp.log(tot) - jnp.log(pos)).reshape(1, 1)

    @pl.when(i == 0)
    def _():
        acc_ref[...] = part

    @pl.when(i != 0)
    def _():
        acc_ref[...] += part


def kernel(z, y, motif_vector):
    y3 = y.reshape(_NBLK, 1, _BLK)
    m2 = -2.0 * motif_vector.reshape(_NC, _NMPC, _NH).transpose(1, 0, 2)
    m2 = m2.reshape(_NM, _NH)
    acc = pl.pallas_call(
        _loss_kernel,
        grid=(_NBLK,),
        in_specs=[
            pl.BlockSpec((_BLK, _NH), lambda i: (i, 0)),
            pl.BlockSpec((1, 1, _BLK), lambda i: (i, 0, 0)),
            pl.BlockSpec((_NM, _NH), lambda i: (0, 0)),
        ],
        out_specs=pl.BlockSpec((1, 1), lambda i: (0, 0)),
        out_shape=jax.ShapeDtypeStruct((1, 1), jnp.float32),
        scratch_shapes=[pltpu.VMEM((1, _NM), jnp.float32)],
    )(z, y3, m2)
    return acc[0, 0] / _B


# speedup vs baseline: 2.0708x; 1.1398x over previous
import jax
import jax.numpy as jnp
from jax.experimental import pallas as pl
from jax.experimental.pallas import tpu as pltpu

_B = 16384
_NH = 256
_NM = 1024
_NC = 128
_NMPC = 8
_EPS = 1e-4

_BLK = 1024
_NBLK = _B // _BLK


def _loss_kernel(z_ref, y_ref, m2_ref, acc_ref, msqe_ref):
    i = pl.program_id(0)

    @pl.when(i == 0)
    def _():
        m2 = m2_ref[...]
        msqe_ref[...] = 0.25 * jnp.sum(m2 * m2, axis=1, keepdims=True).T + _EPS

    zb = z_ref[...]
    zsq = jnp.sum(zb * zb, axis=1, keepdims=True)
    xp2 = jax.lax.dot_general(
        zb, m2_ref[...], (((1,), (1,)), ((), ())),
        preferred_element_type=jnp.float32,
    )
    den = zsq + msqe_ref[...] + xp2
    r = 1.0 + (1.0 - _EPS) / den
    r2 = r * r
    sim = r2 * r2 * r
    persum = sim[:, 0:_NC]
    for k in range(1, _NMPC):
        persum = persum + sim[:, k * _NC:(k + 1) * _NC]
    tot = jnp.sum(persum, axis=1)
    cls = jax.lax.broadcasted_iota(jnp.int32, (_BLK, _NC), 1)
    yb = y_ref[0, 0, :]
    pos = jnp.sum(jnp.where(cls == yb[:, None], persum, 0.0), axis=1)
    part = jnp.sum(jnp.log(tot) - jnp.log(pos)).reshape(1, 1)

    @pl.when(i == 0)
    def _():
        acc_ref[...] = part

    @pl.when(i != 0)
    def _():
        acc_ref[...] += part


def kernel(z, y, motif_vector):
    y3 = y.reshape(_NBLK, 1, _BLK)
    m2 = -2.0 * motif_vector.reshape(_NC, _NMPC, _NH).transpose(1, 0, 2)
    m2 = m2.reshape(_NM, _NH)
    acc = pl.pallas_call(
        _loss_kernel,
        grid=(_NBLK,),
        in_specs=[
            pl.BlockSpec((_BLK, _NH), lambda i: (i, 0)),
            pl.BlockSpec((1, 1, _BLK), lambda i: (i, 0, 0)),
            pl.BlockSpec((_NM, _NH), lambda i: (0, 0)),
        ],
        out_specs=pl.BlockSpec((1, 1), lambda i: (0, 0)),
        out_shape=jax.ShapeDtypeStruct((1, 1), jnp.float32),
        scratch_shapes=[pltpu.VMEM((1, _NM), jnp.float32)],
    )(z, y3, m2)
    return acc[0, 0] / _B
